# baseline (speedup 1.0000x reference)
import functools

import jax
import jax.numpy as jnp
from jax import lax
from jax.experimental import pallas as pl
from jax.experimental.pallas import tpu as pltpu
from jax.experimental.pallas import tpu_sc as plsc

N_VOCAB = 364
D_MODEL = 256
B = 1024
L = 200

NC = 2
NS = 16
NW = NC * NS
B_PER_W = B // NW
LC = (L + 15) // 16
D_CHUNK = 64
N_CHUNKS = D_MODEL // D_CHUNK

_TABLE_WORDS = N_VOCAB * D_MODEL
_IDX_WORDS = B_PER_W * L


def _make_emb_lookup():
  mesh = plsc.VectorSubcoreMesh(core_axis_name="c", subcore_axis_name="s")

  tile_words = D_CHUNK * L

  @functools.partial(
      pl.kernel,
      out_type=jax.ShapeDtypeStruct((B * D_MODEL * L,), jnp.float32),
      mesh=mesh,
      scratch_types=[
          pltpu.VMEM((_TABLE_WORDS,), jnp.float32),
          pltpu.VMEM((_IDX_WORDS + 16,), jnp.int32),
          pltpu.VMEM((tile_words + 16,), jnp.float32),
      ],
      compiler_params=pltpu.CompilerParams(needs_layout_passes=False),
  )
  def emb_lookup(src_hbm, emb_hbm, out_hbm, table_v, idx_v, obuf):
    wid = lax.axis_index("s") * NC + lax.axis_index("c")
    pltpu.sync_copy(emb_hbm, table_v)
    pltpu.sync_copy(
        src_hbm.at[pl.ds(wid * _IDX_WORDS, _IDX_WORDS)],
        idx_v.at[pl.ds(0, _IDX_WORDS)],
    )

    @pl.loop(0, B_PER_W)
    def _batch(b):
      bg = wid * B_PER_W + b
      bases = []
      for lc in range(LC):
        raw = idx_v[pl.ds(b * L + lc * 16, 16)]
        safe = jnp.maximum(jnp.minimum(raw, N_VOCAB - 1), 0)
        bases.append(safe * D_MODEL)

      @pl.loop(0, N_CHUNKS)
      def _chunk(dc):
        d0 = dc * D_CHUNK
        for c in range(D_CHUNK):
          col = d0 + c
          vals = [plsc.load_gather(table_v, [bases[lc] + col])
                  for lc in range(LC)]
          for lc in range(LC):
            obuf[pl.ds(c * L + lc * 16, 16)] = vals[lc]
        pltpu.sync_copy(
            obuf.at[pl.ds(0, tile_words)],
            out_hbm.at[pl.ds(bg * (D_MODEL * L) + d0 * L, tile_words)],
        )

  return emb_lookup


_emb_lookup = None


def kernel(src_seq, x_lengths, emb):
  del x_lengths
  global _emb_lookup
  if _emb_lookup is None:
    _emb_lookup = _make_emb_lookup()
  src_flat = src_seq.reshape(-1).astype(jnp.int32)
  emb_flat = emb.reshape(-1)
  out_flat = _emb_lookup(src_flat, emb_flat)
  return out_flat.reshape(B, D_MODEL, L)


# speedup vs baseline: 5.4262x; 5.4262x over previous
import functools

import jax
import jax.numpy as jnp
from jax import lax
from jax.experimental import pallas as pl
from jax.experimental.pallas import tpu as pltpu
from jax.experimental.pallas import tpu_sc as plsc

N_VOCAB = 364
D_MODEL = 256
B = 1024
L = 200

NC = 2
NS = 16
NW = NC * NS
ROWS = B * L
ROWS_PER_W = ROWS // NW
CHUNK = 128
N_CHUNKS = ROWS_PER_W // CHUNK


def _make_emb_lookup():
  mesh = plsc.VectorSubcoreMesh(core_axis_name="c", subcore_axis_name="s")

  @functools.partial(
      pl.kernel,
      out_type=jax.ShapeDtypeStruct((ROWS, D_MODEL), jnp.float32),
      mesh=mesh,
      scratch_types=[
          pltpu.VMEM((ROWS_PER_W,), jnp.int32),
          pltpu.VMEM((CHUNK, D_MODEL), jnp.float32),
          pltpu.VMEM((CHUNK, D_MODEL), jnp.float32),
          pltpu.SemaphoreType.DMA,
          pltpu.SemaphoreType.DMA,
          pltpu.SemaphoreType.DMA,
          pltpu.SemaphoreType.DMA,
      ],
      compiler_params=pltpu.CompilerParams(needs_layout_passes=False),
  )
  def emb_lookup(src_hbm, emb_hbm, out_hbm, idx_v, buf_a, buf_b,
                 sem_ga, sem_gb, sem_sa, sem_sb):
    wid = lax.axis_index("s") * NC + lax.axis_index("c")
    row0 = wid * ROWS_PER_W
    pltpu.sync_copy(src_hbm.at[pl.ds(wid * ROWS_PER_W, ROWS_PER_W)], idx_v)

    def gather(k, buf, sem):
      pltpu.async_copy(emb_hbm.at[idx_v.at[pl.ds(k * CHUNK, CHUNK)]], buf, sem)

    def scatter(k, buf, sem):
      pltpu.async_copy(buf, out_hbm.at[pl.ds(row0 + k * CHUNK, CHUNK)], sem)

    def wait(src, dst, sem):
      pltpu.make_async_copy(src, dst, sem).wait()

    def wait_gather(buf, sem):
      wait(emb_hbm.at[idx_v.at[pl.ds(0, CHUNK)]], buf, sem)

    def wait_scatter(buf, sem):
      wait(buf, out_hbm.at[pl.ds(row0, CHUNK)], sem)

    gather(0, buf_a, sem_ga)

    @pl.loop(0, N_CHUNKS, step=2)
    def _pair(k):
      @pl.when(k > 0)
      def _():
        wait_scatter(buf_b, sem_sb)
      gather(k + 1, buf_b, sem_gb)
      wait_gather(buf_a, sem_ga)
      scatter(k, buf_a, sem_sa)
      wait_scatter(buf_a, sem_sa)
      @pl.when(k + 2 < N_CHUNKS)
      def _():
        gather(k + 2, buf_a, sem_ga)
      wait_gather(buf_b, sem_gb)
      scatter(k + 1, buf_b, sem_sb)

    wait_scatter(buf_b, sem_sb)

  return emb_lookup


_emb_lookup = None


def kernel(src_seq, x_lengths, emb):
  del x_lengths
  global _emb_lookup
  if _emb_lookup is None:
    _emb_lookup = _make_emb_lookup()
  src_flat = src_seq.reshape(-1).astype(jnp.int32)
  rows = _emb_lookup(src_flat, emb)
  return rows.reshape(B, L, D_MODEL).transpose(0, 2, 1)
